# baseline (speedup 1.0000x reference)
import functools

import jax
import jax.numpy as jnp
from jax.experimental import pallas as pl
from jax.experimental.pallas import tpu as pltpu

NUM_EMB = 1024
EMB = 64
NH = 128
INC = 3
CC = 0.25

HW = 56 * 56
CBLK = HW


def _conv(x, w, b, stride, pad):
    y = jax.lax.conv_general_dilated(x, w, (stride, stride), [(pad, pad), (pad, pad)],
                                     dimension_numbers=('NCHW', 'OIHW', 'NCHW'))
    return y + b[None, :, None, None]


def _convT(x, w, b, stride, pad):
    k = w.shape[2]
    w2 = jnp.transpose(jnp.flip(w, (2, 3)), (1, 0, 2, 3))
    p = k - 1 - pad
    y = jax.lax.conv_general_dilated(x, w2, (1, 1), [(p, p), (p, p)],
                                     lhs_dilation=(stride, stride),
                                     dimension_numbers=('NCHW', 'OIHW', 'NCHW'))
    return y + b[None, :, None, None]


def _convT_small(a, w, b):
    C, K = w.shape[0], w.shape[1]
    w48 = jnp.transpose(w, (2, 3, 1, 0)).reshape(16 * K, C)[:, :, None, None]
    o = jax.lax.conv_general_dilated(a, w48, (1, 1), [(0, 0), (0, 0)],
                                     dimension_numbers=('NCHW', 'OIHW', 'NCHW'))
    n, h = a.shape[0], a.shape[2]
    op = jnp.pad(o, ((0, 0), (0, 0), (1, 1), (1, 1)))
    taps = {0: ((1, 0), (3, -1)), 1: ((2, 0), (0, 1))}
    phases = []
    for py in (0, 1):
        for px in (0, 1):
            acc = None
            for ky, sy in taps[py]:
                for kx, sx in taps[px]:
                    g = ky * 4 + kx
                    part = jax.lax.slice(
                        op, (0, g * K, 1 + sy, 1 + sx),
                        (n, (g + 1) * K, 1 + sy + h, 1 + sx + h))
                    acc = part if acc is None else acc + part
            phases.append(acc)
    s = jnp.stack(phases, axis=-1).reshape(n, K, h, h, 2, 2)
    s = jnp.transpose(s, (0, 1, 2, 4, 3, 5)).reshape(n, K, 2 * h, 2 * h)
    return s + b[None, :, None, None]


def _convT_phase(a, w, b):
    K = w.shape[1]
    subs = []
    for py in (0, 1):
        for px in (0, 1):
            ky = jnp.array([3 - py, 1 - py])
            kx = jnp.array([3 - px, 1 - px])
            sub = w[:, :, ky][:, :, :, kx]
            subs.append(jnp.transpose(sub, (1, 0, 2, 3)))
    wcat = jnp.concatenate(subs, axis=0)
    o = jax.lax.conv_general_dilated(a, wcat, (1, 1), [(1, 1), (1, 1)],
                                     dimension_numbers=('NCHW', 'OIHW', 'NCHW'))
    n, h = a.shape[0], a.shape[2]
    phases = []
    for p, (py, px) in enumerate([(0, 0), (0, 1), (1, 0), (1, 1)]):
        phases.append(jax.lax.slice(o, (0, p * K, py, px),
                                    (n, (p + 1) * K, py + h, px + h)))
    s = jnp.stack(phases, axis=-1).reshape(n, K, h, h, 2, 2)
    s = jnp.transpose(s, (0, 1, 2, 4, 3, 5)).reshape(n, K, 2 * h, 2 * h)
    return s + b[None, :, None, None]


def _vq_body(z_ref, cb_ref, idx_ref, q_ref, dsum_ref):
    b = pl.program_id(0)
    c = pl.program_id(1)

    zb = z_ref[0]
    cb = cb_ref[:]

    cb_norm2 = jnp.sum(cb * cb, axis=1)
    prod = jax.lax.dot_general(cb, zb, (((1,), (0,)), ((), ())),
                               preferred_element_type=jnp.float32)
    scores = cb_norm2[:, None] - 2.0 * prod

    idx = jnp.argmin(scores, axis=0).astype(jnp.int32)
    smin = jnp.min(scores, axis=0)
    idx_ref[0, 0, :] = idx

    onehot = (jax.lax.broadcasted_iota(jnp.int32, (NUM_EMB, CBLK), 0)
              == idx[None, :]).astype(jnp.float32)
    q_ref[0] = jax.lax.dot_general(cb, onehot, (((0,), (0,)), ((), ())),
                                   precision=jax.lax.Precision.HIGHEST,
                                   preferred_element_type=jnp.float32)

    z_norm2 = jnp.sum(zb * zb, axis=0)
    part = jnp.sum(smin + z_norm2)

    @pl.when(jnp.logical_and(b == 0, c == 0))
    def _():
        dsum_ref[0, 0] = 0.0

    dsum_ref[0, 0] += part


@functools.partial(jax.jit, static_argnames=('interpret',))
def _vq(z3, codebook, interpret=False):
    nb = z3.shape[0]
    ncb = HW // CBLK
    idx, q, dsum = pl.pallas_call(
        _vq_body,
        grid=(nb, ncb),
        in_specs=[
            pl.BlockSpec((1, EMB, CBLK), lambda b, c: (b, 0, c)),
            pl.BlockSpec((NUM_EMB, EMB), lambda b, c: (0, 0)),
        ],
        out_specs=[
            pl.BlockSpec((1, 1, CBLK), lambda b, c: (b, 0, c)),
            pl.BlockSpec((1, EMB, CBLK), lambda b, c: (b, 0, c)),
            pl.BlockSpec((1, 1), lambda b, c: (0, 0),
                         memory_space=pltpu.MemorySpace.SMEM),
        ],
        out_shape=[
            jax.ShapeDtypeStruct((nb, 1, HW), jnp.int32),
            jax.ShapeDtypeStruct((nb, EMB, HW), jnp.float32),
            jax.ShapeDtypeStruct((1, 1), jnp.float32),
        ],
        interpret=interpret,
    )(z3, codebook)
    return idx, q, dsum


def kernel(x, enc_w1, enc_b1, enc_w2, enc_b2, enc_w3, enc_b3, codebook,
           dec_w1, dec_b1, dec_w2, dec_b2, dec_w3, dec_b3):
    h = jax.nn.relu(_conv(x, enc_w1, enc_b1, 2, 1))
    h = jax.nn.relu(_conv(h, enc_w2, enc_b2, 2, 1))
    z = _conv(h, enc_w3, enc_b3, 1, 1)

    nb = z.shape[0]
    z3 = z.reshape(nb, EMB, HW)
    idx, q, dsum = _vq(z3, codebook)

    vq_loss = (1.0 + CC) * dsum[0, 0] / (nb * HW * EMB)
    quantized = q.reshape(nb, EMB, 56, 56)

    h = jax.nn.relu(_conv(quantized, dec_w1, dec_b1, 1, 1))
    h = jax.nn.relu(_convT(h, dec_w2, dec_b2, 2, 1))
    x_recon = jax.nn.sigmoid(_convT_small(h, dec_w3, dec_b3))
    return (vq_loss, x_recon, idx.reshape(nb * HW)[:, None])


# speedup vs baseline: 1.1912x; 1.1912x over previous
import functools

import jax
import jax.numpy as jnp
from jax import lax
from jax.experimental import pallas as pl
from jax.experimental.pallas import tpu as pltpu
from jax.experimental.pallas import tpu_sc as plsc

NUM_EMB = 1024
EMB = 64
NH = 128
INC = 3
CC = 0.25

HW = 56 * 56
NB = 8
B_TOT = NB * HW
NW = 32
BPW = B_TOT // NW


def _conv(x, w, b, stride, pad):
    y = jax.lax.conv_general_dilated(x, w, (stride, stride), [(pad, pad), (pad, pad)],
                                     dimension_numbers=('NCHW', 'OIHW', 'NCHW'))
    return y + b[None, :, None, None]


def _convT(x, w, b, stride, pad):
    k = w.shape[2]
    w2 = jnp.transpose(jnp.flip(w, (2, 3)), (1, 0, 2, 3))
    p = k - 1 - pad
    y = jax.lax.conv_general_dilated(x, w2, (1, 1), [(p, p), (p, p)],
                                     lhs_dilation=(stride, stride),
                                     dimension_numbers=('NCHW', 'OIHW', 'NCHW'))
    return y + b[None, :, None, None]


def _vq_body(z_ref, cb_ref, idx_ref, dsum_ref):
    b = pl.program_id(0)

    zb = z_ref[0]
    cb = cb_ref[:]

    cb_norm2 = jnp.sum(cb * cb, axis=1)
    prod = jax.lax.dot_general(cb, zb, (((1,), (0,)), ((), ())),
                               preferred_element_type=jnp.float32)
    scores = cb_norm2[:, None] - 2.0 * prod

    idx = jnp.argmin(scores, axis=0).astype(jnp.int32)
    smin = jnp.min(scores, axis=0)
    idx_ref[0, 0, :] = idx

    z_norm2 = jnp.sum(zb * zb, axis=0)
    part = jnp.sum(smin + z_norm2)

    @pl.when(b == 0)
    def _():
        dsum_ref[0, 0] = 0.0

    dsum_ref[0, 0] += part


@functools.partial(jax.jit, static_argnames=('interpret',))
def _vq(z3, codebook, interpret=False):
    nb = z3.shape[0]
    idx, dsum = pl.pallas_call(
        _vq_body,
        grid=(nb,),
        in_specs=[
            pl.BlockSpec((1, EMB, HW), lambda b: (b, 0, 0)),
            pl.BlockSpec((NUM_EMB, EMB), lambda b: (0, 0)),
        ],
        out_specs=[
            pl.BlockSpec((1, 1, HW), lambda b: (b, 0, 0)),
            pl.BlockSpec((1, 1), lambda b: (0, 0),
                         memory_space=pltpu.MemorySpace.SMEM),
        ],
        out_shape=[
            jax.ShapeDtypeStruct((nb, 1, HW), jnp.int32),
            jax.ShapeDtypeStruct((1, 1), jnp.float32),
        ],
        interpret=interpret,
    )(z3, codebook)
    return idx, dsum


_sc_mesh = plsc.VectorSubcoreMesh(core_axis_name="c", subcore_axis_name="s")

EMBP = 128


@functools.partial(
    pl.kernel,
    mesh=_sc_mesh,
    out_type=jax.ShapeDtypeStruct((B_TOT, EMBP), jnp.float32),
    scratch_types=[
        pltpu.VMEM((BPW,), jnp.int32),
        pltpu.VMEM((BPW, EMBP), jnp.float32),
        pltpu.SemaphoreType.DMA,
    ],
)
def _sc_dequant(cb_hbm, idx_hbm, out_hbm, idx_v, rows_v, sem):
    wid = lax.axis_index("s") * 2 + lax.axis_index("c")
    base = wid * BPW
    pltpu.sync_copy(idx_hbm.at[pl.ds(base, BPW)], idx_v)
    pltpu.async_copy(cb_hbm.at[idx_v], rows_v, sem).wait()
    pltpu.sync_copy(rows_v, out_hbm.at[pl.ds(base, BPW)])


def kernel(x, enc_w1, enc_b1, enc_w2, enc_b2, enc_w3, enc_b3, codebook,
           dec_w1, dec_b1, dec_w2, dec_b2, dec_w3, dec_b3):
    h = jax.nn.relu(_conv(x, enc_w1, enc_b1, 2, 1))
    h = jax.nn.relu(_conv(h, enc_w2, enc_b2, 2, 1))
    z = _conv(h, enc_w3, enc_b3, 1, 1)

    nb = z.shape[0]
    z3 = z.reshape(nb, EMB, HW)
    idx, dsum = _vq(z3, codebook)

    vq_loss = (1.0 + CC) * dsum[0, 0] / (nb * HW * EMB)

    idx_flat = idx.reshape(B_TOT)
    cb_pad = jnp.pad(codebook, ((0, 0), (0, EMBP - EMB)))
    q = _sc_dequant(cb_pad, idx_flat)[:, :EMB]
    quantized = jnp.transpose(q.reshape(nb, 56, 56, EMB), (0, 3, 1, 2))

    h = jax.nn.relu(_conv(quantized, dec_w1, dec_b1, 1, 1))
    h = jax.nn.relu(_convT(h, dec_w2, dec_b2, 2, 1))
    x_recon = jax.nn.sigmoid(_convT(h, dec_w3, dec_b3, 2, 1))
    return (vq_loss, x_recon, idx_flat[:, None])


# speedup vs baseline: 1.6569x; 1.3909x over previous
import functools

import jax
import jax.numpy as jnp
from jax.experimental import pallas as pl
from jax.experimental.pallas import tpu as pltpu

NUM_EMB = 1024
EMB = 64
NH = 128
INC = 3
CC = 0.25

HW = 56 * 56
CBLK = HW


def _conv(x, w, b, stride, pad):
    y = jax.lax.conv_general_dilated(x, w, (stride, stride), [(pad, pad), (pad, pad)],
                                     dimension_numbers=('NCHW', 'OIHW', 'NCHW'))
    return y + b[None, :, None, None]


def _convT(x, w, b, stride, pad):
    k = w.shape[2]
    w2 = jnp.transpose(jnp.flip(w, (2, 3)), (1, 0, 2, 3))
    p = k - 1 - pad
    y = jax.lax.conv_general_dilated(x, w2, (1, 1), [(p, p), (p, p)],
                                     lhs_dilation=(stride, stride),
                                     dimension_numbers=('NCHW', 'OIHW', 'NCHW'))
    return y + b[None, :, None, None]


def _vq_body(z_ref, cb_ref, idx_ref, q_ref, dsum_ref):
    b = pl.program_id(0)
    c = pl.program_id(1)

    zb = z_ref[0]
    cb = cb_ref[:]

    cb_norm2 = jnp.sum(cb * cb, axis=1)
    prod = jax.lax.dot_general(cb, zb, (((1,), (0,)), ((), ())),
                               preferred_element_type=jnp.float32)
    scores = cb_norm2[:, None] - 2.0 * prod

    idx = jnp.argmin(scores, axis=0).astype(jnp.int32)
    smin = jnp.min(scores, axis=0)
    idx_ref[0, 0, :] = idx

    onehot = (jax.lax.broadcasted_iota(jnp.int32, (NUM_EMB, CBLK), 0)
              == idx[None, :]).astype(jnp.float32)
    q_ref[0] = jax.lax.dot_general(cb, onehot, (((0,), (0,)), ((), ())),
                                   precision=jax.lax.Precision.HIGHEST,
                                   preferred_element_type=jnp.float32)

    z_norm2 = jnp.sum(zb * zb, axis=0)
    part = jnp.sum(smin + z_norm2)

    @pl.when(jnp.logical_and(b == 0, c == 0))
    def _():
        dsum_ref[0, 0] = 0.0

    dsum_ref[0, 0] += part


@functools.partial(jax.jit, static_argnames=('interpret',))
def _vq(z3, codebook, interpret=False):
    nb = z3.shape[0]
    ncb = HW // CBLK
    idx, q, dsum = pl.pallas_call(
        _vq_body,
        grid=(nb, ncb),
        in_specs=[
            pl.BlockSpec((1, EMB, CBLK), lambda b, c: (b, 0, c)),
            pl.BlockSpec((NUM_EMB, EMB), lambda b, c: (0, 0)),
        ],
        out_specs=[
            pl.BlockSpec((1, 1, CBLK), lambda b, c: (b, 0, c)),
            pl.BlockSpec((1, EMB, CBLK), lambda b, c: (b, 0, c)),
            pl.BlockSpec((1, 1), lambda b, c: (0, 0),
                         memory_space=pltpu.MemorySpace.SMEM),
        ],
        out_shape=[
            jax.ShapeDtypeStruct((nb, 1, HW), jnp.int32),
            jax.ShapeDtypeStruct((nb, EMB, HW), jnp.float32),
            jax.ShapeDtypeStruct((1, 1), jnp.float32),
        ],
        interpret=interpret,
    )(z3, codebook)
    return idx, q, dsum


def kernel(x, enc_w1, enc_b1, enc_w2, enc_b2, enc_w3, enc_b3, codebook,
           dec_w1, dec_b1, dec_w2, dec_b2, dec_w3, dec_b3):
    h = jax.nn.relu(_conv(x, enc_w1, enc_b1, 2, 1))
    h = jax.nn.relu(_conv(h, enc_w2, enc_b2, 2, 1))
    z = _conv(h, enc_w3, enc_b3, 1, 1)

    nb = z.shape[0]
    z3 = z.reshape(nb, EMB, HW)
    idx, q, dsum = _vq(z3, codebook)

    vq_loss = (1.0 + CC) * dsum[0, 0] / (nb * HW * EMB)
    quantized = q.reshape(nb, EMB, 56, 56)

    h = jax.nn.relu(_conv(quantized, dec_w1, dec_b1, 1, 1))
    h = jax.nn.relu(_convT(h, dec_w2, dec_b2, 2, 1))
    hn = jnp.transpose(h, (0, 2, 3, 1))
    w2 = jnp.transpose(jnp.flip(dec_w3, (2, 3)), (2, 3, 0, 1))
    y = jax.lax.conv_general_dilated(hn, w2, (1, 1), [(2, 2), (2, 2)],
                                     lhs_dilation=(2, 2),
                                     dimension_numbers=('NHWC', 'HWIO', 'NHWC'))
    x_recon = jax.nn.sigmoid(jnp.transpose(y + dec_b3, (0, 3, 1, 2)))
    return (vq_loss, x_recon, idx.reshape(nb * HW)[:, None])
